# baseline (speedup 1.0000x reference)
import functools

import jax
import jax.numpy as jnp
from jax import lax
from jax.experimental import pallas as pl
from jax.experimental.pallas import tpu as pltpu
from jax.experimental.pallas import tpu_sc as plsc

N_SRC = 10000
N_DST = 10000
E = 320000
H = 128

NC = 2
NS = 16
NW = NC * NS
NP = 10240
EP = 327680
EW = EP // NW
GRP = 128
G = EW // GRP
RPT = NP // NS

_mesh = plsc.VectorSubcoreMesh(core_axis_name="c", subcore_axis_name="s")


@functools.partial(
    pl.kernel,
    out_type=jax.ShapeDtypeStruct((NC, 2, NP), jnp.float32),
    mesh=_mesh,
    scratch_types=[
        pltpu.VMEM((G, GRP), jnp.int32),
        pltpu.VMEM((G, GRP), jnp.int32),
        pltpu.VMEM((GRP,), jnp.float32),
        pltpu.VMEM_SHARED((NP,), jnp.float32),
        pltpu.VMEM_SHARED((NP,), jnp.float32),
    ],
)
def _sc_degrees(src_hbm, dst_hbm, zeros_hbm, deg_out, sidx, didx, ones_v,
                degs_sh, degd_sh):
    c = lax.axis_index("c")
    s = lax.axis_index("s")
    wid = s * NC + c
    pltpu.sync_copy(src_hbm.at[wid], sidx)
    pltpu.sync_copy(dst_hbm.at[wid], didx)
    for i in range(GRP // 16):
        ones_v[pl.ds(i * 16, 16)] = jnp.ones((16,), jnp.float32)
    pltpu.sync_copy(zeros_hbm.at[pl.ds(s * RPT, RPT)],
                    degs_sh.at[pl.ds(s * RPT, RPT)])
    pltpu.sync_copy(zeros_hbm.at[pl.ds(s * RPT, RPT)],
                    degd_sh.at[pl.ds(s * RPT, RPT)])
    plsc.subcore_barrier()

    def body(g, carry):
        pltpu.sync_copy(ones_v, degs_sh.at[sidx.at[g]], add=True)
        pltpu.sync_copy(ones_v, degd_sh.at[didx.at[g]], add=True)
        return carry

    lax.fori_loop(0, G, body, 0)
    plsc.subcore_barrier()
    pltpu.sync_copy(degs_sh.at[pl.ds(s * RPT, RPT)],
                    deg_out.at[c, 0, pl.ds(s * RPT, RPT)])
    pltpu.sync_copy(degd_sh.at[pl.ds(s * RPT, RPT)],
                    deg_out.at[c, 1, pl.ds(s * RPT, RPT)])


RB = 2
IB = 4
NG0 = 80
NG1 = 80
TOTG = NS * (NG0 + NG1)


@functools.partial(
    pl.kernel,
    out_type=jax.ShapeDtypeStruct((NC, NP, H), jnp.float32),
    mesh=_mesh,
    scratch_types=[
        pltpu.VMEM((IB, 2, GRP), jnp.int32),
        pltpu.VMEM((RB, GRP, H), jnp.float32),
        pltpu.SemaphoreType.DMA((IB,)),
        pltpu.SemaphoreType.DMA((RB,)),
        pltpu.SemaphoreType.DMA((RB,)),
        pltpu.VMEM_SHARED((NP, H), jnp.float32),
    ],
)
def _sc_gather(featn_hbm, idx_hbm, zeros_hbm, agg_out, ichunk, rows_v,
               isem, gsem, ssem, agg_sh):
    c = lax.axis_index("c")
    s = lax.axis_index("s")
    ng = jnp.where(c == 0, NG0, NG1)
    rowbase = jnp.where(c == 0, s * NG0, NS * NG0 + s * NG1)
    pltpu.sync_copy(idx_hbm.at[pl.ds(rowbase, IB)], ichunk)
    pltpu.sync_copy(zeros_hbm, agg_sh.at[pl.ds(s * RPT, RPT)])
    plsc.subcore_barrier()

    for b in range(RB):
        pltpu.async_copy(featn_hbm.at[ichunk.at[b, 0]], rows_v.at[b],
                         gsem.at[b])

    def step(g, carry):
        b = lax.rem(g, RB)
        sl = lax.rem(g, IB)
        pltpu.make_async_copy(featn_hbm.at[ichunk.at[sl, 0]], rows_v.at[b],
                              gsem.at[b]).wait()
        pltpu.async_copy(rows_v.at[b], agg_sh.at[ichunk.at[sl, 1]],
                         ssem.at[b], add=True)
        pltpu.make_async_copy(rows_v.at[b], agg_sh.at[ichunk.at[sl, 1]],
                              ssem.at[b]).wait()

        sl2 = lax.rem(g + RB, IB)

        @pl.when(jnp.logical_and(g + RB >= IB, g + RB < ng))
        def _():
            pltpu.make_async_copy(idx_hbm.at[rowbase + g + RB],
                                  ichunk.at[sl2], isem.at[sl2]).wait()

        @pl.when(g + RB < ng)
        def _():
            pltpu.async_copy(featn_hbm.at[ichunk.at[sl2, 0]], rows_v.at[b],
                             gsem.at[b])

        @pl.when(g + IB < ng)
        def _():
            pltpu.async_copy(idx_hbm.at[rowbase + g + IB], ichunk.at[sl],
                             isem.at[sl])

        return carry

    lax.fori_loop(0, ng, step, 0)
    plsc.subcore_barrier()
    pltpu.sync_copy(agg_sh.at[pl.ds(s * RPT, RPT)],
                    agg_out.at[c, pl.ds(s * RPT, RPT)])


def _prescale_body(feat_ref, degs_ref, degd_ref, featn_ref, normd_ref):
    ns = lax.rsqrt(jnp.maximum(degs_ref[...], 1.0))
    featn_ref[...] = feat_ref[...] * ns
    normd_ref[...] = lax.rsqrt(jnp.maximum(degd_ref[...], 1.0))


_prescale = pl.pallas_call(
    _prescale_body,
    out_shape=(
        jax.ShapeDtypeStruct((NP, H), jnp.float32),
        jax.ShapeDtypeStruct((NP, 1), jnp.float32),
    ),
)

BD = 1024


def _tail_body(aggp_ref, normd_ref, t0_ref, Wc_ref, bc_ref, W1_ref, b1_ref,
               Wa_ref, Wo_ref, bo_ref, wl1_ref, bl1_ref, wlo_ref, blo_ref,
               out_ref):
    agg = (aggp_ref[0] + aggp_ref[1]) * normd_ref[...]
    z1 = jnp.dot(agg, Wc_ref[...], preferred_element_type=jnp.float32) + bc_ref[...]
    s1 = jnp.where(z1 >= 0, z1, 0.01 * z1)
    t1 = jnp.tanh(jnp.dot(s1, W1_ref[...], preferred_element_type=jnp.float32)
                  + b1_ref[...])
    t2 = jnp.tanh(t0_ref[...] * wl1_ref[...] + bl1_ref[...])
    e1 = jnp.sum(t1 * Wa_ref[...], axis=1, keepdims=True)
    e2 = jnp.sum(t2 * Wa_ref[...], axis=1, keepdims=True)
    m = jnp.maximum(e1, e2)
    a1 = jnp.exp(e1 - m)
    a2 = jnp.exp(e2 - m)
    o1 = jnp.sum(s1 * Wo_ref[...], axis=1, keepdims=True)
    o2 = t0_ref[...] * wlo_ref[...] + blo_ref[...]
    out_ref[...] = (a1 * o1 + a2 * o2) / (a1 + a2) + bo_ref[...]


_tail = pl.pallas_call(
    _tail_body,
    grid=(NP // BD,),
    in_specs=[
        pl.BlockSpec((NC, BD, H), lambda i: (0, i, 0)),
        pl.BlockSpec((BD, 1), lambda i: (i, 0)),
        pl.BlockSpec((BD, 1), lambda i: (i, 0)),
        pl.BlockSpec((H, H), lambda i: (0, 0)),
        pl.BlockSpec((1, H), lambda i: (0, 0)),
        pl.BlockSpec((H, 2 * H), lambda i: (0, 0)),
        pl.BlockSpec((1, 2 * H), lambda i: (0, 0)),
        pl.BlockSpec((1, 2 * H), lambda i: (0, 0)),
        pl.BlockSpec((1, H), lambda i: (0, 0)),
        pl.BlockSpec((1, 1), lambda i: (0, 0)),
        pl.BlockSpec((1, 2 * H), lambda i: (0, 0)),
        pl.BlockSpec((1, 2 * H), lambda i: (0, 0)),
        pl.BlockSpec((1, 1), lambda i: (0, 0)),
        pl.BlockSpec((1, 1), lambda i: (0, 0)),
    ],
    out_specs=pl.BlockSpec((BD, 1), lambda i: (i, 0)),
    out_shape=jax.ShapeDtypeStruct((NP, 1), jnp.float32),
)


def kernel(x, edge_index, dst_t0, W_conv, b_conv, W_lin, b_lin, W1, b1, Wa,
           ba, Wo, bo):
    del ba
    feat = x[:, 0, 0, :]
    feat_pad = jnp.pad(feat, ((0, NP - N_SRC), (0, 0)))
    pad_idx = jnp.full((EP - E,), NP - 1, jnp.int32)
    srcp = jnp.concatenate([edge_index[0], pad_idx])
    dstp = jnp.concatenate([edge_index[1], pad_idx])
    src_p = srcp.reshape(NW, G, GRP)
    dst_p = dstp.reshape(NW, G, GRP)
    idx_all = jnp.stack([srcp, dstp], 0).reshape(2, TOTG, GRP).transpose(1, 0, 2)

    deg_parts = _sc_degrees(src_p, dst_p, jnp.zeros((NP,), jnp.float32))
    degs = (deg_parts[0, 0] + deg_parts[1, 0]).reshape(NP, 1)
    degd = (deg_parts[0, 1] + deg_parts[1, 1]).reshape(NP, 1)

    featn, normd = _prescale(feat_pad, degs, degd)

    agg_parts = _sc_gather(featn, idx_all,
                           jnp.zeros((RPT, H), jnp.float32))

    t0_pad = jnp.pad(dst_t0, ((0, NP - N_DST), (0, 0)))
    wl1 = (W_lin @ W1).reshape(1, 2 * H)
    bl1 = (b_lin.reshape(1, H) @ W1 + b1).reshape(1, 2 * H)
    wlo = (W_lin @ Wo).reshape(1, 1)
    blo = (b_lin.reshape(1, H) @ Wo).reshape(1, 1)
    out = _tail(agg_parts, normd, t0_pad,
                W_conv, b_conv.reshape(1, H),
                W1, b1.reshape(1, 2 * H),
                Wa.reshape(1, 2 * H),
                Wo.reshape(1, H), bo.reshape(1, 1),
                wl1, bl1, wlo, blo)
    return out[:N_DST]


# speedup vs baseline: 1.0781x; 1.0781x over previous
import functools

import jax
import jax.numpy as jnp
from jax import lax
from jax.experimental import pallas as pl
from jax.experimental.pallas import tpu as pltpu
from jax.experimental.pallas import tpu_sc as plsc

N_SRC = 10000
N_DST = 10000
E = 320000
H = 128

NC = 2
NS = 16
NW = NC * NS
NP = 10240
EP = 327680
EW = EP // NW
GRP = 128
G = EW // GRP
RPT = NP // NS

_mesh = plsc.VectorSubcoreMesh(core_axis_name="c", subcore_axis_name="s")


@functools.partial(
    pl.kernel,
    out_type=jax.ShapeDtypeStruct((NC, 2, NP), jnp.float32),
    mesh=_mesh,
    scratch_types=[
        pltpu.VMEM((G, GRP), jnp.int32),
        pltpu.VMEM((G, GRP), jnp.int32),
        pltpu.VMEM((GRP,), jnp.float32),
        pltpu.VMEM_SHARED((NP,), jnp.float32),
        pltpu.VMEM_SHARED((NP,), jnp.float32),
    ],
)
def _sc_degrees(src_hbm, dst_hbm, zeros_hbm, deg_out, sidx, didx, ones_v,
                degs_sh, degd_sh):
    c = lax.axis_index("c")
    s = lax.axis_index("s")
    wid = s * NC + c
    pltpu.sync_copy(src_hbm.at[wid], sidx)
    pltpu.sync_copy(dst_hbm.at[wid], didx)
    for i in range(GRP // 16):
        ones_v[pl.ds(i * 16, 16)] = jnp.ones((16,), jnp.float32)
    pltpu.sync_copy(zeros_hbm.at[pl.ds(s * RPT, RPT)],
                    degs_sh.at[pl.ds(s * RPT, RPT)])
    pltpu.sync_copy(zeros_hbm.at[pl.ds(s * RPT, RPT)],
                    degd_sh.at[pl.ds(s * RPT, RPT)])
    plsc.subcore_barrier()

    def body(g, carry):
        pltpu.sync_copy(ones_v, degs_sh.at[sidx.at[g]], add=True)
        pltpu.sync_copy(ones_v, degd_sh.at[didx.at[g]], add=True)
        return carry

    lax.fori_loop(0, G, body, 0)
    plsc.subcore_barrier()
    pltpu.sync_copy(degs_sh.at[pl.ds(s * RPT, RPT)],
                    deg_out.at[c, 0, pl.ds(s * RPT, RPT)])
    pltpu.sync_copy(degd_sh.at[pl.ds(s * RPT, RPT)],
                    deg_out.at[c, 1, pl.ds(s * RPT, RPT)])


RB = 2
IB = 4
NG0 = 144
NG1 = 16
TOTG = NS * (NG0 + NG1)


@functools.partial(
    pl.kernel,
    out_type=jax.ShapeDtypeStruct((NC, NP, H), jnp.float32),
    mesh=_mesh,
    scratch_types=[
        pltpu.VMEM((IB, 2, GRP), jnp.int32),
        pltpu.VMEM((RB, GRP, H), jnp.float32),
        pltpu.SemaphoreType.DMA((IB,)),
        pltpu.SemaphoreType.DMA((RB,)),
        pltpu.SemaphoreType.DMA((RB,)),
        pltpu.VMEM_SHARED((NP, H), jnp.float32),
    ],
)
def _sc_gather(featn_hbm, idx_hbm, zeros_hbm, agg_out, ichunk, rows_v,
               isem, gsem, ssem, agg_sh):
    c = lax.axis_index("c")
    s = lax.axis_index("s")
    ng = jnp.where(c == 0, NG0, NG1)
    rowbase = jnp.where(c == 0, s * NG0, NS * NG0 + s * NG1)
    pltpu.sync_copy(idx_hbm.at[pl.ds(rowbase, IB)], ichunk)
    pltpu.sync_copy(zeros_hbm, agg_sh.at[pl.ds(s * RPT, RPT)])
    plsc.subcore_barrier()

    for b in range(RB):
        pltpu.async_copy(featn_hbm.at[ichunk.at[b, 0]], rows_v.at[b],
                         gsem.at[b])

    def step(g, carry):
        b = lax.rem(g, RB)
        sl = lax.rem(g, IB)
        pltpu.make_async_copy(featn_hbm.at[ichunk.at[sl, 0]], rows_v.at[b],
                              gsem.at[b]).wait()
        pltpu.async_copy(rows_v.at[b], agg_sh.at[ichunk.at[sl, 1]],
                         ssem.at[b], add=True)
        pltpu.make_async_copy(rows_v.at[b], agg_sh.at[ichunk.at[sl, 1]],
                              ssem.at[b]).wait()

        sl2 = lax.rem(g + RB, IB)

        @pl.when(jnp.logical_and(g + RB >= IB, g + RB < ng))
        def _():
            pltpu.make_async_copy(idx_hbm.at[rowbase + g + RB],
                                  ichunk.at[sl2], isem.at[sl2]).wait()

        @pl.when(g + RB < ng)
        def _():
            pltpu.async_copy(featn_hbm.at[ichunk.at[sl2, 0]], rows_v.at[b],
                             gsem.at[b])

        @pl.when(g + IB < ng)
        def _():
            pltpu.async_copy(idx_hbm.at[rowbase + g + IB], ichunk.at[sl],
                             isem.at[sl])

        return carry

    lax.fori_loop(0, ng, step, 0)
    plsc.subcore_barrier()
    pltpu.sync_copy(agg_sh.at[pl.ds(s * RPT, RPT)],
                    agg_out.at[c, pl.ds(s * RPT, RPT)])


def _prescale_body(feat_ref, degs_ref, degd_ref, featn_ref, normd_ref):
    ns = lax.rsqrt(jnp.maximum(degs_ref[...], 1.0))
    featn_ref[...] = feat_ref[...] * ns
    normd_ref[...] = lax.rsqrt(jnp.maximum(degd_ref[...], 1.0))


_prescale = pl.pallas_call(
    _prescale_body,
    out_shape=(
        jax.ShapeDtypeStruct((NP, H), jnp.float32),
        jax.ShapeDtypeStruct((NP, 1), jnp.float32),
    ),
)

BD = 1024


def _tail_body(aggp_ref, normd_ref, t0_ref, Wc_ref, bc_ref, W1_ref, b1_ref,
               Wa_ref, Wo_ref, bo_ref, wl1_ref, bl1_ref, wlo_ref, blo_ref,
               out_ref):
    agg = (aggp_ref[0] + aggp_ref[1]) * normd_ref[...]
    z1 = jnp.dot(agg, Wc_ref[...], preferred_element_type=jnp.float32) + bc_ref[...]
    s1 = jnp.where(z1 >= 0, z1, 0.01 * z1)
    t1 = jnp.tanh(jnp.dot(s1, W1_ref[...], preferred_element_type=jnp.float32)
                  + b1_ref[...])
    t2 = jnp.tanh(t0_ref[...] * wl1_ref[...] + bl1_ref[...])
    e1 = jnp.sum(t1 * Wa_ref[...], axis=1, keepdims=True)
    e2 = jnp.sum(t2 * Wa_ref[...], axis=1, keepdims=True)
    m = jnp.maximum(e1, e2)
    a1 = jnp.exp(e1 - m)
    a2 = jnp.exp(e2 - m)
    o1 = jnp.sum(s1 * Wo_ref[...], axis=1, keepdims=True)
    o2 = t0_ref[...] * wlo_ref[...] + blo_ref[...]
    out_ref[...] = (a1 * o1 + a2 * o2) / (a1 + a2) + bo_ref[...]


_tail = pl.pallas_call(
    _tail_body,
    grid=(NP // BD,),
    in_specs=[
        pl.BlockSpec((NC, BD, H), lambda i: (0, i, 0)),
        pl.BlockSpec((BD, 1), lambda i: (i, 0)),
        pl.BlockSpec((BD, 1), lambda i: (i, 0)),
        pl.BlockSpec((H, H), lambda i: (0, 0)),
        pl.BlockSpec((1, H), lambda i: (0, 0)),
        pl.BlockSpec((H, 2 * H), lambda i: (0, 0)),
        pl.BlockSpec((1, 2 * H), lambda i: (0, 0)),
        pl.BlockSpec((1, 2 * H), lambda i: (0, 0)),
        pl.BlockSpec((1, H), lambda i: (0, 0)),
        pl.BlockSpec((1, 1), lambda i: (0, 0)),
        pl.BlockSpec((1, 2 * H), lambda i: (0, 0)),
        pl.BlockSpec((1, 2 * H), lambda i: (0, 0)),
        pl.BlockSpec((1, 1), lambda i: (0, 0)),
        pl.BlockSpec((1, 1), lambda i: (0, 0)),
    ],
    out_specs=pl.BlockSpec((BD, 1), lambda i: (i, 0)),
    out_shape=jax.ShapeDtypeStruct((NP, 1), jnp.float32),
)


def kernel(x, edge_index, dst_t0, W_conv, b_conv, W_lin, b_lin, W1, b1, Wa,
           ba, Wo, bo):
    del ba
    feat = x[:, 0, 0, :]
    feat_pad = jnp.pad(feat, ((0, NP - N_SRC), (0, 0)))
    pad_idx = jnp.full((EP - E,), NP - 1, jnp.int32)
    srcp = jnp.concatenate([edge_index[0], pad_idx])
    dstp = jnp.concatenate([edge_index[1], pad_idx])
    src_p = srcp.reshape(NW, G, GRP)
    dst_p = dstp.reshape(NW, G, GRP)
    idx_all = jnp.stack([srcp, dstp], 0).reshape(2, TOTG, GRP).transpose(1, 0, 2)

    deg_parts = _sc_degrees(src_p, dst_p, jnp.zeros((NP,), jnp.float32))
    degs = (deg_parts[0, 0] + deg_parts[1, 0]).reshape(NP, 1)
    degd = (deg_parts[0, 1] + deg_parts[1, 1]).reshape(NP, 1)

    featn, normd = _prescale(feat_pad, degs, degd)

    agg_parts = _sc_gather(featn, idx_all,
                           jnp.zeros((RPT, H), jnp.float32))

    t0_pad = jnp.pad(dst_t0, ((0, NP - N_DST), (0, 0)))
    wl1 = (W_lin @ W1).reshape(1, 2 * H)
    bl1 = (b_lin.reshape(1, H) @ W1 + b1).reshape(1, 2 * H)
    wlo = (W_lin @ Wo).reshape(1, 1)
    blo = (b_lin.reshape(1, H) @ Wo).reshape(1, 1)
    out = _tail(agg_parts, normd, t0_pad,
                W_conv, b_conv.reshape(1, H),
                W1, b1.reshape(1, 2 * H),
                Wa.reshape(1, 2 * H),
                Wo.reshape(1, H), bo.reshape(1, 1),
                wl1, bl1, wlo, blo)
    return out[:N_DST]


# speedup vs baseline: 1.0801x; 1.0019x over previous
import functools

import jax
import jax.numpy as jnp
from jax import lax
from jax.experimental import pallas as pl
from jax.experimental.pallas import tpu as pltpu
from jax.experimental.pallas import tpu_sc as plsc

N_SRC = 10000
N_DST = 10000
E = 320000
H = 128

NC = 2
NS = 16
NW = NC * NS
NP = 10240
EP = 327680
EW = EP // NW
GRP = 128
G = EW // GRP
RPT = NP // NS

_mesh = plsc.VectorSubcoreMesh(core_axis_name="c", subcore_axis_name="s")


@functools.partial(
    pl.kernel,
    out_type=jax.ShapeDtypeStruct((NC, 2, NP), jnp.float32),
    mesh=_mesh,
    scratch_types=[
        pltpu.VMEM((G, GRP), jnp.int32),
        pltpu.VMEM((G, GRP), jnp.int32),
        pltpu.VMEM((GRP,), jnp.float32),
        pltpu.VMEM_SHARED((NP,), jnp.float32),
        pltpu.VMEM_SHARED((NP,), jnp.float32),
    ],
)
def _sc_degrees(src_hbm, dst_hbm, zeros_hbm, deg_out, sidx, didx, ones_v,
                degs_sh, degd_sh):
    c = lax.axis_index("c")
    s = lax.axis_index("s")
    wid = s * NC + c
    pltpu.sync_copy(src_hbm.at[wid], sidx)
    pltpu.sync_copy(dst_hbm.at[wid], didx)
    for i in range(GRP // 16):
        ones_v[pl.ds(i * 16, 16)] = jnp.ones((16,), jnp.float32)
    pltpu.sync_copy(zeros_hbm.at[pl.ds(s * RPT, RPT)],
                    degs_sh.at[pl.ds(s * RPT, RPT)])
    pltpu.sync_copy(zeros_hbm.at[pl.ds(s * RPT, RPT)],
                    degd_sh.at[pl.ds(s * RPT, RPT)])
    plsc.subcore_barrier()

    def body(g, carry):
        pltpu.sync_copy(ones_v, degs_sh.at[sidx.at[g]], add=True)
        pltpu.sync_copy(ones_v, degd_sh.at[didx.at[g]], add=True)
        return carry

    lax.fori_loop(0, G, body, 0)
    plsc.subcore_barrier()
    pltpu.sync_copy(degs_sh.at[pl.ds(s * RPT, RPT)],
                    deg_out.at[c, 0, pl.ds(s * RPT, RPT)])
    pltpu.sync_copy(degd_sh.at[pl.ds(s * RPT, RPT)],
                    deg_out.at[c, 1, pl.ds(s * RPT, RPT)])


RB = 2
IB = 4
NG0 = 152
NG1 = 8
TOTG = NS * (NG0 + NG1)


@functools.partial(
    pl.kernel,
    out_type=jax.ShapeDtypeStruct((NC, NP, H), jnp.float32),
    mesh=_mesh,
    scratch_types=[
        pltpu.VMEM((IB, 2, GRP), jnp.int32),
        pltpu.VMEM((RB, GRP, H), jnp.float32),
        pltpu.SemaphoreType.DMA((IB,)),
        pltpu.SemaphoreType.DMA((RB,)),
        pltpu.SemaphoreType.DMA((RB,)),
        pltpu.VMEM_SHARED((NP, H), jnp.float32),
    ],
)
def _sc_gather(featn_hbm, idx_hbm, zeros_hbm, agg_out, ichunk, rows_v,
               isem, gsem, ssem, agg_sh):
    c = lax.axis_index("c")
    s = lax.axis_index("s")
    ng = jnp.where(c == 0, NG0, NG1)
    rowbase = jnp.where(c == 0, s * NG0, NS * NG0 + s * NG1)
    pltpu.sync_copy(idx_hbm.at[pl.ds(rowbase, IB)], ichunk)
    pltpu.sync_copy(zeros_hbm, agg_sh.at[pl.ds(s * RPT, RPT)])
    plsc.subcore_barrier()

    for b in range(RB):
        pltpu.async_copy(featn_hbm.at[ichunk.at[b, 0]], rows_v.at[b],
                         gsem.at[b])

    def step(g, carry):
        b = lax.rem(g, RB)
        sl = lax.rem(g, IB)
        pltpu.make_async_copy(featn_hbm.at[ichunk.at[sl, 0]], rows_v.at[b],
                              gsem.at[b]).wait()
        pltpu.async_copy(rows_v.at[b], agg_sh.at[ichunk.at[sl, 1]],
                         ssem.at[b], add=True)
        pltpu.make_async_copy(rows_v.at[b], agg_sh.at[ichunk.at[sl, 1]],
                              ssem.at[b]).wait()

        sl2 = lax.rem(g + RB, IB)

        @pl.when(jnp.logical_and(g + RB >= IB, g + RB < ng))
        def _():
            pltpu.make_async_copy(idx_hbm.at[rowbase + g + RB],
                                  ichunk.at[sl2], isem.at[sl2]).wait()

        @pl.when(g + RB < ng)
        def _():
            pltpu.async_copy(featn_hbm.at[ichunk.at[sl2, 0]], rows_v.at[b],
                             gsem.at[b])

        @pl.when(g + IB < ng)
        def _():
            pltpu.async_copy(idx_hbm.at[rowbase + g + IB], ichunk.at[sl],
                             isem.at[sl])

        return carry

    lax.fori_loop(0, ng, step, 0)
    plsc.subcore_barrier()
    pltpu.sync_copy(agg_sh.at[pl.ds(s * RPT, RPT)],
                    agg_out.at[c, pl.ds(s * RPT, RPT)])


def _prescale_body(feat_ref, degs_ref, degd_ref, featn_ref, normd_ref):
    ns = lax.rsqrt(jnp.maximum(degs_ref[...], 1.0))
    featn_ref[...] = feat_ref[...] * ns
    normd_ref[...] = lax.rsqrt(jnp.maximum(degd_ref[...], 1.0))


_prescale = pl.pallas_call(
    _prescale_body,
    out_shape=(
        jax.ShapeDtypeStruct((NP, H), jnp.float32),
        jax.ShapeDtypeStruct((NP, 1), jnp.float32),
    ),
)

BD = 1024


def _tail_body(aggp_ref, normd_ref, t0_ref, Wc_ref, bc_ref, W1_ref, b1_ref,
               Wa_ref, Wo_ref, bo_ref, wl1_ref, bl1_ref, wlo_ref, blo_ref,
               out_ref):
    agg = (aggp_ref[0] + aggp_ref[1]) * normd_ref[...]
    z1 = jnp.dot(agg, Wc_ref[...], preferred_element_type=jnp.float32) + bc_ref[...]
    s1 = jnp.where(z1 >= 0, z1, 0.01 * z1)
    t1 = jnp.tanh(jnp.dot(s1, W1_ref[...], preferred_element_type=jnp.float32)
                  + b1_ref[...])
    t2 = jnp.tanh(t0_ref[...] * wl1_ref[...] + bl1_ref[...])
    e1 = jnp.sum(t1 * Wa_ref[...], axis=1, keepdims=True)
    e2 = jnp.sum(t2 * Wa_ref[...], axis=1, keepdims=True)
    m = jnp.maximum(e1, e2)
    a1 = jnp.exp(e1 - m)
    a2 = jnp.exp(e2 - m)
    o1 = jnp.sum(s1 * Wo_ref[...], axis=1, keepdims=True)
    o2 = t0_ref[...] * wlo_ref[...] + blo_ref[...]
    out_ref[...] = (a1 * o1 + a2 * o2) / (a1 + a2) + bo_ref[...]


_tail = pl.pallas_call(
    _tail_body,
    grid=(NP // BD,),
    in_specs=[
        pl.BlockSpec((NC, BD, H), lambda i: (0, i, 0)),
        pl.BlockSpec((BD, 1), lambda i: (i, 0)),
        pl.BlockSpec((BD, 1), lambda i: (i, 0)),
        pl.BlockSpec((H, H), lambda i: (0, 0)),
        pl.BlockSpec((1, H), lambda i: (0, 0)),
        pl.BlockSpec((H, 2 * H), lambda i: (0, 0)),
        pl.BlockSpec((1, 2 * H), lambda i: (0, 0)),
        pl.BlockSpec((1, 2 * H), lambda i: (0, 0)),
        pl.BlockSpec((1, H), lambda i: (0, 0)),
        pl.BlockSpec((1, 1), lambda i: (0, 0)),
        pl.BlockSpec((1, 2 * H), lambda i: (0, 0)),
        pl.BlockSpec((1, 2 * H), lambda i: (0, 0)),
        pl.BlockSpec((1, 1), lambda i: (0, 0)),
        pl.BlockSpec((1, 1), lambda i: (0, 0)),
    ],
    out_specs=pl.BlockSpec((BD, 1), lambda i: (i, 0)),
    out_shape=jax.ShapeDtypeStruct((NP, 1), jnp.float32),
)


def kernel(x, edge_index, dst_t0, W_conv, b_conv, W_lin, b_lin, W1, b1, Wa,
           ba, Wo, bo):
    del ba
    feat = x[:, 0, 0, :]
    feat_pad = jnp.pad(feat, ((0, NP - N_SRC), (0, 0)))
    pad_idx = jnp.full((EP - E,), NP - 1, jnp.int32)
    srcp = jnp.concatenate([edge_index[0], pad_idx])
    dstp = jnp.concatenate([edge_index[1], pad_idx])
    src_p = srcp.reshape(NW, G, GRP)
    dst_p = dstp.reshape(NW, G, GRP)
    idx_all = jnp.stack([srcp, dstp], 0).reshape(2, TOTG, GRP).transpose(1, 0, 2)

    deg_parts = _sc_degrees(src_p, dst_p, jnp.zeros((NP,), jnp.float32))
    degs = (deg_parts[0, 0] + deg_parts[1, 0]).reshape(NP, 1)
    degd = (deg_parts[0, 1] + deg_parts[1, 1]).reshape(NP, 1)

    featn, normd = _prescale(feat_pad, degs, degd)

    agg_parts = _sc_gather(featn, idx_all,
                           jnp.zeros((RPT, H), jnp.float32))

    t0_pad = jnp.pad(dst_t0, ((0, NP - N_DST), (0, 0)))
    wl1 = (W_lin @ W1).reshape(1, 2 * H)
    bl1 = (b_lin.reshape(1, H) @ W1 + b1).reshape(1, 2 * H)
    wlo = (W_lin @ Wo).reshape(1, 1)
    blo = (b_lin.reshape(1, H) @ Wo).reshape(1, 1)
    out = _tail(agg_parts, normd, t0_pad,
                W_conv, b_conv.reshape(1, H),
                W1, b1.reshape(1, 2 * H),
                Wa.reshape(1, 2 * H),
                Wo.reshape(1, H), bo.reshape(1, 1),
                wl1, bl1, wlo, blo)
    return out[:N_DST]
